# baseline (speedup 1.0000x reference)
import functools

import jax
import jax.numpy as jnp
from jax import lax
from jax.experimental import pallas as pl
from jax.experimental.pallas import tpu as pltpu
from jax.experimental.pallas import tpu_sc as plsc

N = 10000
E = 320000
D = 128
H = 128
C = 64
WEIGHT_DECAY = 5e-4

_NC = 2
_NS = 16
_NW = _NC * _NS


def _make_spmm(n, d, e, k):
    per_w = e // _NW
    n_chunks = per_w // k
    n_pairs = n_chunks // 2
    zr = 40
    rchunks = n // zr
    riters = (rchunks + _NS - 1) // _NS
    mesh = plsc.VectorSubcoreMesh(core_axis_name="c", subcore_axis_name="s")

    @functools.partial(
        pl.kernel,
        out_type=jax.ShapeDtypeStruct((_NC * n, d), jnp.float32),
        mesh=mesh,
        scratch_types=[
            pltpu.VMEM((4, k), jnp.int32),
            pltpu.VMEM((4, k), jnp.int32),
            pltpu.VMEM((4, k), jnp.float32),
            pltpu.VMEM((4, k, d), jnp.float32),
            pltpu.VMEM_SHARED((n, d), jnp.float32),
            pltpu.SemaphoreType.DMA((4,)),
            pltpu.SemaphoreType.DMA((4,)),
            pltpu.SemaphoreType.DMA((4,)),
            pltpu.SemaphoreType.DMA((4,)),
            pltpu.SemaphoreType.DMA((4,)),
        ],
    )
    def spmm(h_hbm, src_hbm, dst_hbm, w_hbm, out_hbm,
             sb, db, wb, rows, accum,
             sem_g, sem_s, sem_src, sem_w, sem_d):
        c = lax.axis_index("c")
        s = lax.axis_index("s")
        wid = c * _NS + s
        base0 = wid * per_w

        def srcload(ci, b):
            pltpu.async_copy(src_hbm.at[pl.ds(base0 + ci * k, k)],
                             sb.at[b], sem_src.at[b])

        def wload(ci, b):
            pltpu.async_copy(w_hbm.at[pl.ds(base0 + ci * k, k)],
                             wb.at[b], sem_w.at[b])

        def dload(ci, b):
            pltpu.async_copy(dst_hbm.at[pl.ds(base0 + ci * k, k)],
                             db.at[b], sem_d.at[b])

        def gissue(b):
            pltpu.async_copy(h_hbm.at[sb.at[b]], rows.at[b], sem_g.at[b])

        def sissue(b):
            pltpu.async_copy(rows.at[b], accum.at[db.at[b]],
                             sem_s.at[b], add=True)

        def wait_g(b):
            pltpu.make_async_copy(h_hbm.at[sb.at[b]], rows.at[b],
                                  sem_g.at[b]).wait()

        def wait_s(b):
            pltpu.make_async_copy(h_hbm.at[sb.at[b]], rows.at[b],
                                  sem_s.at[b]).wait()

        def wait_small(hbm, vm, b, sem):
            pltpu.make_async_copy(hbm.at[pl.ds(0, k)], vm.at[b], sem.at[b]).wait()

        def scale(b, ci):
            @plsc.parallel_loop(0, k // 16)
            def cgroup(gi):
                w16 = wb[b, pl.ds(gi * 16, 16)]
                for l in range(16):
                    i = gi * 16 + l
                    bc = lax.gather(
                        w16, jnp.full((16, 1), l, jnp.int32),
                        lax.GatherDimensionNumbers(
                            offset_dims=(), collapsed_slice_dims=(0,),
                            start_index_map=(0,)),
                        slice_sizes=(1,),
                        mode=lax.GatherScatterMode.PROMISE_IN_BOUNDS)
                    for j in range(d // 16):
                        rows[b, i, pl.ds(j * 16, 16)] = (
                            rows[b, i, pl.ds(j * 16, 16)] * bc)

        def slot(b, ci):
            wait_g(b)
            wait_small(w_hbm, wb, b, sem_w)

            @pl.when(ci + 4 < n_chunks)
            def _():
                srcload(ci + 4, b)
            scale(b, ci)

            @pl.when(ci + 4 < n_chunks)
            def _():
                wload(ci + 4, b)
            wait_small(dst_hbm, db, b, sem_d)
            sissue(b)
            cn = ci + 2
            pb = (b + 2) % 4

            @pl.when(cn < n_chunks)
            def _():
                @pl.when(cn >= 4)
                def _():
                    wait_s(pb)
                dload(cn, pb)
                wait_small(src_hbm, sb, pb, sem_src)
                gissue(pb)

        for b in range(4):
            srcload(b, b)
            wload(b, b)
        dload(0, 0)
        dload(1, 1)

        for i in range(zr):
            for j in range(d // 16):
                rows[3, i, pl.ds(j * 16, 16)] = jnp.zeros((16,), jnp.float32)
        zstage = rows.at[3, pl.ds(0, zr)]

        def zcopy(t, issue):
            ch = t * _NS + s
            cp = pltpu.make_async_copy(
                zstage, accum.at[pl.ds(ch * zr, zr)], sem_s.at[t % 4])
            if issue:
                cp.start()
            else:
                cp.wait()

        for t in range(riters):
            if t == riters - 1:
                @pl.when(s < rchunks - (riters - 1) * _NS)
                def _():
                    zcopy(riters - 1, True)
            else:
                zcopy(t, True)

        for b in range(2):
            wait_small(src_hbm, sb, b, sem_src)
            gissue(b)

        for t in range(riters):
            if t == riters - 1:
                @pl.when(s < rchunks - (riters - 1) * _NS)
                def _():
                    zcopy(riters - 1, False)
            else:
                zcopy(t, False)
        plsc.subcore_barrier()

        ngroups = n_chunks // 4
        tail = n_chunks - ngroups * 4

        def gbody(g, carry):
            for b in range(4):
                slot(b, g * 4 + b)
            return carry
        lax.fori_loop(0, ngroups, gbody, 0)
        for t in range(tail):
            ci = ngroups * 4 + t
            b = ci % 4
            wait_g(b)
            wait_small(w_hbm, wb, b, sem_w)
            scale(b, ci)
            wait_small(dst_hbm, db, b, sem_d)
            sissue(b)
        for b in range(4):
            wait_s(b)
        plsc.subcore_barrier()

        def ocopy(t, issue):
            ch = t * _NS + s
            off = ch * zr
            cp = pltpu.make_async_copy(
                accum.at[pl.ds(off, zr)],
                out_hbm.at[pl.ds(c * n + off, zr)], sem_g.at[t % 4])
            if issue:
                cp.start()
            else:
                cp.wait()

        for t in range(riters):
            if t == riters - 1:
                @pl.when(s < rchunks - (riters - 1) * _NS)
                def _():
                    ocopy(riters - 1, True)
            else:
                ocopy(t, True)
        for t in range(riters):
            if t == riters - 1:
                @pl.when(s < rchunks - (riters - 1) * _NS)
                def _():
                    ocopy(riters - 1, False)
            else:
                ocopy(t, False)

    return spmm


_spmm_h = _make_spmm(N, H, E, 80)


def _mm1_body(x_ref, w_ref, o_ref):
    o_ref[...] = jnp.dot(x_ref[...], w_ref[...],
                         preferred_element_type=jnp.float32)


def _mm2_body(p_ref, w_ref, o_ref):
    h = jnp.maximum(p_ref[:N] + p_ref[N:], 0.0)
    wp = jnp.concatenate(
        [w_ref[...], jnp.zeros((H, H - C), jnp.float32)], axis=1)
    o_ref[...] = jnp.dot(h, wp, preferred_element_type=jnp.float32)


def _loss_body(q_ref, lab_ref, mask_ref, w1_ref, o_ref):
    q = q_ref[:N, :C] + q_ref[N:, :C]
    lab = lab_ref[...]
    mx = jnp.max(q, axis=-1, keepdims=True)
    lse = mx + jnp.log(jnp.sum(jnp.exp(q - mx), axis=-1, keepdims=True))
    mk = mask_ref[...]
    s1 = jnp.sum(lab * (lse - q) * mk)
    s2 = jnp.sum(mk)
    l2 = 0.5 * WEIGHT_DECAY * jnp.sum(w1_ref[...] * w1_ref[...])
    o_ref[0, 0] = s1 / s2 + l2


def kernel(x, label, mask, edge_index, edge_weight, W1, W2):
    src = edge_index[0]
    dst = edge_index[1]

    h = pl.pallas_call(
        _mm1_body,
        out_shape=jax.ShapeDtypeStruct((N, H), jnp.float32),
    )(x, W1)

    p = _spmm_h(h, src, dst, edge_weight)

    g = pl.pallas_call(
        _mm2_body,
        out_shape=jax.ShapeDtypeStruct((N, H), jnp.float32),
    )(p, W2)

    q = _spmm_h(g, src, dst, edge_weight)

    loss = pl.pallas_call(
        _loss_body,
        out_shape=jax.ShapeDtypeStruct((1, 1), jnp.float32),
        out_specs=pl.BlockSpec(memory_space=pltpu.SMEM),
    )(q, label, jnp.reshape(mask, (N, 1)), W1)

    return loss[0, 0]


# speedup vs baseline: 1.2040x; 1.2040x over previous
import functools

import jax
import jax.numpy as jnp
from jax import lax
from jax.experimental import pallas as pl
from jax.experimental.pallas import tpu as pltpu
from jax.experimental.pallas import tpu_sc as plsc

N = 10000
E = 320000
D = 128
H = 128
C = 64
WEIGHT_DECAY = 5e-4

_NC = 2
_NS = 16
_NW = _NC * _NS


def _make_spmm(n, d, e, k):
    per_w = e // _NW
    n_chunks = per_w // k
    n_pairs = n_chunks // 2
    zr = 40
    rchunks = n // zr
    riters = (rchunks + _NS - 1) // _NS
    mesh = plsc.VectorSubcoreMesh(core_axis_name="c", subcore_axis_name="s")

    @functools.partial(
        pl.kernel,
        out_type=jax.ShapeDtypeStruct((_NC * n, d), jnp.float32),
        mesh=mesh,
        scratch_types=[
            pltpu.VMEM((4, k), jnp.int32),
            pltpu.VMEM((4, k), jnp.int32),
            pltpu.VMEM((4, k), jnp.float32),
            pltpu.VMEM((4, k, d), jnp.float32),
            pltpu.VMEM_SHARED((n, d), jnp.float32),
            pltpu.SemaphoreType.DMA((4,)),
            pltpu.SemaphoreType.DMA((4,)),
            pltpu.SemaphoreType.DMA((4,)),
            pltpu.SemaphoreType.DMA((4,)),
            pltpu.SemaphoreType.DMA((4,)),
        ],
    )
    def spmm(h_hbm, src_hbm, dst_hbm, w_hbm, out_hbm,
             sb, db, wb, rows, accum,
             sem_g, sem_s, sem_src, sem_w, sem_d):
        c = lax.axis_index("c")
        s = lax.axis_index("s")
        wid = c * _NS + s
        base0 = wid * per_w

        def srcload(ci, b):
            pltpu.async_copy(src_hbm.at[pl.ds(base0 + ci * k, k)],
                             sb.at[b], sem_src.at[b])

        def wload(ci, b):
            pltpu.async_copy(w_hbm.at[pl.ds(base0 + ci * k, k)],
                             wb.at[b], sem_w.at[b])

        def dload(ci, b):
            pltpu.async_copy(dst_hbm.at[pl.ds(base0 + ci * k, k)],
                             db.at[b], sem_d.at[b])

        def gissue(b):
            pltpu.async_copy(h_hbm.at[sb.at[b]], rows.at[b], sem_g.at[b])

        def sissue(b):
            pltpu.async_copy(rows.at[b], accum.at[db.at[b]],
                             sem_s.at[b], add=True)

        def wait_g(b):
            pltpu.make_async_copy(h_hbm.at[sb.at[b]], rows.at[b],
                                  sem_g.at[b]).wait()

        def wait_s(b):
            pltpu.make_async_copy(h_hbm.at[sb.at[b]], rows.at[b],
                                  sem_s.at[b]).wait()

        def wait_small(hbm, vm, b, sem):
            pltpu.make_async_copy(hbm.at[pl.ds(0, k)], vm.at[b], sem.at[b]).wait()

        def scale(b, ci):
            def cgroup(gi, carry):
                w16 = wb[b, pl.ds(gi * 16, 16)]
                for l in range(16):
                    i = gi * 16 + l
                    bc = lax.gather(
                        w16, jnp.full((16, 1), l, jnp.int32),
                        lax.GatherDimensionNumbers(
                            offset_dims=(), collapsed_slice_dims=(0,),
                            start_index_map=(0,)),
                        slice_sizes=(1,),
                        mode=lax.GatherScatterMode.PROMISE_IN_BOUNDS)
                    for j in range(d // 16):
                        rows[b, i, pl.ds(j * 16, 16)] = (
                            rows[b, i, pl.ds(j * 16, 16)] * bc)
                return carry
            lax.fori_loop(0, k // 16, cgroup, 0)

        def slot(b, ci):
            wait_g(b)
            wait_small(w_hbm, wb, b, sem_w)

            @pl.when(ci + 4 < n_chunks)
            def _():
                srcload(ci + 4, b)
            scale(b, ci)

            @pl.when(ci + 4 < n_chunks)
            def _():
                wload(ci + 4, b)
            wait_small(dst_hbm, db, b, sem_d)
            sissue(b)
            cn = ci + 2
            pb = (b + 2) % 4

            @pl.when(cn < n_chunks)
            def _():
                @pl.when(cn >= 4)
                def _():
                    wait_s(pb)
                dload(cn, pb)
                wait_small(src_hbm, sb, pb, sem_src)
                gissue(pb)

        for b in range(4):
            srcload(b, b)
            wload(b, b)
        dload(0, 0)
        dload(1, 1)

        for i in range(zr):
            for j in range(d // 16):
                rows[3, i, pl.ds(j * 16, 16)] = jnp.zeros((16,), jnp.float32)
        zstage = rows.at[3, pl.ds(0, zr)]

        def zcopy(t, issue):
            ch = t * _NS + s
            cp = pltpu.make_async_copy(
                zstage, accum.at[pl.ds(ch * zr, zr)], sem_s.at[t % 4])
            if issue:
                cp.start()
            else:
                cp.wait()

        for t in range(riters):
            if t == riters - 1:
                @pl.when(s < rchunks - (riters - 1) * _NS)
                def _():
                    zcopy(riters - 1, True)
            else:
                zcopy(t, True)

        for b in range(2):
            wait_small(src_hbm, sb, b, sem_src)
            gissue(b)

        for t in range(riters):
            if t == riters - 1:
                @pl.when(s < rchunks - (riters - 1) * _NS)
                def _():
                    zcopy(riters - 1, False)
            else:
                zcopy(t, False)
        plsc.subcore_barrier()

        ngroups = n_chunks // 4
        tail = n_chunks - ngroups * 4

        def gbody(g, carry):
            for b in range(4):
                slot(b, g * 4 + b)
            return carry
        lax.fori_loop(0, ngroups, gbody, 0)
        for t in range(tail):
            ci = ngroups * 4 + t
            b = ci % 4
            wait_g(b)
            wait_small(w_hbm, wb, b, sem_w)
            scale(b, ci)
            wait_small(dst_hbm, db, b, sem_d)
            sissue(b)
        for b in range(4):
            wait_s(b)
        plsc.subcore_barrier()

        def ocopy(t, issue):
            ch = t * _NS + s
            off = ch * zr
            cp = pltpu.make_async_copy(
                accum.at[pl.ds(off, zr)],
                out_hbm.at[pl.ds(c * n + off, zr)], sem_g.at[t % 4])
            if issue:
                cp.start()
            else:
                cp.wait()

        for t in range(riters):
            if t == riters - 1:
                @pl.when(s < rchunks - (riters - 1) * _NS)
                def _():
                    ocopy(riters - 1, True)
            else:
                ocopy(t, True)
        for t in range(riters):
            if t == riters - 1:
                @pl.when(s < rchunks - (riters - 1) * _NS)
                def _():
                    ocopy(riters - 1, False)
            else:
                ocopy(t, False)

    return spmm


_spmm_h = _make_spmm(N, H, E, 80)


def _mm1_body(x_ref, w_ref, o_ref):
    o_ref[...] = jnp.dot(x_ref[...], w_ref[...],
                         preferred_element_type=jnp.float32)


def _mm2_body(p_ref, w_ref, o_ref):
    h = jnp.maximum(p_ref[:N] + p_ref[N:], 0.0)
    wp = jnp.concatenate(
        [w_ref[...], jnp.zeros((H, H - C), jnp.float32)], axis=1)
    o_ref[...] = jnp.dot(h, wp, preferred_element_type=jnp.float32)


def _loss_body(q_ref, lab_ref, mask_ref, w1_ref, o_ref):
    q = q_ref[:N, :C] + q_ref[N:, :C]
    lab = lab_ref[...]
    mx = jnp.max(q, axis=-1, keepdims=True)
    lse = mx + jnp.log(jnp.sum(jnp.exp(q - mx), axis=-1, keepdims=True))
    mk = mask_ref[...]
    s1 = jnp.sum(lab * (lse - q) * mk)
    s2 = jnp.sum(mk)
    l2 = 0.5 * WEIGHT_DECAY * jnp.sum(w1_ref[...] * w1_ref[...])
    o_ref[0, 0] = s1 / s2 + l2


def kernel(x, label, mask, edge_index, edge_weight, W1, W2):
    src = edge_index[0]
    dst = edge_index[1]

    h = pl.pallas_call(
        _mm1_body,
        out_shape=jax.ShapeDtypeStruct((N, H), jnp.float32),
    )(x, W1)

    p = _spmm_h(h, src, dst, edge_weight)

    g = pl.pallas_call(
        _mm2_body,
        out_shape=jax.ShapeDtypeStruct((N, H), jnp.float32),
    )(p, W2)

    q = _spmm_h(g, src, dst, edge_weight)

    loss = pl.pallas_call(
        _loss_body,
        out_shape=jax.ShapeDtypeStruct((1, 1), jnp.float32),
        out_specs=pl.BlockSpec(memory_space=pltpu.SMEM),
    )(q, label, jnp.reshape(mask, (N, 1)), W1)

    return loss[0, 0]


# speedup vs baseline: 1.2266x; 1.0187x over previous
import functools

import jax
import jax.numpy as jnp
from jax import lax
from jax.experimental import pallas as pl
from jax.experimental.pallas import tpu as pltpu
from jax.experimental.pallas import tpu_sc as plsc

N = 10000
E = 320000
D = 128
H = 128
C = 64
WEIGHT_DECAY = 5e-4

_NC = 2
_NS = 16
_NW = _NC * _NS


def _make_spmm(n, d, e, k):
    per_w = e // _NW
    n_chunks = per_w // k
    tail_e = per_w - n_chunks * k
    zr = 40
    rchunks = n // zr
    riters = (rchunks + _NS - 1) // _NS
    mesh = plsc.VectorSubcoreMesh(core_axis_name="c", subcore_axis_name="s")

    @functools.partial(
        pl.kernel,
        out_type=jax.ShapeDtypeStruct((_NC * n, d), jnp.float32),
        mesh=mesh,
        scratch_types=[
            pltpu.VMEM((3, k), jnp.int32),
            pltpu.VMEM((3, k), jnp.int32),
            pltpu.VMEM((3, k), jnp.float32),
            pltpu.VMEM((3, k, d), jnp.float32),
            pltpu.VMEM((1, 16), jnp.int32),
            pltpu.VMEM_SHARED((n, d), jnp.float32),
            pltpu.SemaphoreType.DMA((3,)),
            pltpu.SemaphoreType.DMA((3,)),
            pltpu.SemaphoreType.DMA((3,)),
            pltpu.SemaphoreType.DMA((3,)),
            pltpu.SemaphoreType.DMA((3,)),
        ],
    )
    def spmm(h_hbm, src_hbm, dst_hbm, w_hbm, out_hbm,
             sb, db, wb, rows, dt, accum,
             sem_g, sem_s, sem_src, sem_w, sem_d):
        c = lax.axis_index("c")
        s = lax.axis_index("s")
        wid = c * _NS + s
        base0 = wid * per_w

        def srcload(ci, b):
            pltpu.async_copy(src_hbm.at[pl.ds(base0 + ci * k, k)],
                             sb.at[b], sem_src.at[b])

        def wload(ci, b):
            pltpu.async_copy(w_hbm.at[pl.ds(base0 + ci * k, k)],
                             wb.at[b], sem_w.at[b])

        def dload(ci, b):
            pltpu.async_copy(dst_hbm.at[pl.ds(base0 + ci * k, k)],
                             db.at[b], sem_d.at[b])

        def gissue(b):
            pltpu.async_copy(h_hbm.at[sb.at[b]], rows.at[b], sem_g.at[b])

        def sissue(b):
            pltpu.async_copy(rows.at[b], accum.at[db.at[b]],
                             sem_s.at[b], add=True)

        def wait_g(b):
            pltpu.make_async_copy(h_hbm.at[sb.at[b]], rows.at[b],
                                  sem_g.at[b]).wait()

        def wait_s(b):
            pltpu.make_async_copy(h_hbm.at[sb.at[b]], rows.at[b],
                                  sem_s.at[b]).wait()

        def wait_small(hbm, vm, b, sem):
            pltpu.make_async_copy(hbm.at[pl.ds(0, k)], vm.at[b], sem.at[b]).wait()

        def scale(b, ci):
            def cgroup(gi, carry):
                w16 = wb[b, pl.ds(gi * 16, 16)]
                for l in range(16):
                    i = gi * 16 + l
                    bc = lax.gather(
                        w16, jnp.full((16, 1), l, jnp.int32),
                        lax.GatherDimensionNumbers(
                            offset_dims=(), collapsed_slice_dims=(0,),
                            start_index_map=(0,)),
                        slice_sizes=(1,),
                        mode=lax.GatherScatterMode.PROMISE_IN_BOUNDS)
                    for j in range(d // 16):
                        rows[b, i, pl.ds(j * 16, 16)] = (
                            rows[b, i, pl.ds(j * 16, 16)] * bc)
                return carry
            lax.fori_loop(0, k // 16, cgroup, 0)

        def slot(b, ci):
            wait_g(b)
            wait_small(w_hbm, wb, b, sem_w)

            @pl.when(ci + 3 < n_chunks)
            def _():
                srcload(ci + 3, b)
            scale(b, ci)

            @pl.when(ci + 3 < n_chunks)
            def _():
                wload(ci + 3, b)
            wait_small(dst_hbm, db, b, sem_d)
            sissue(b)
            cn = ci + 2
            pb = (b + 2) % 3

            @pl.when(cn < n_chunks)
            def _():
                @pl.when(cn >= 3)
                def _():
                    wait_s(pb)
                dload(cn, pb)
                wait_small(src_hbm, sb, pb, sem_src)
                gissue(pb)

        for b in range(3):
            srcload(b, b)
            wload(b, b)
        dload(0, 0)
        dload(1, 1)

        for i in range(zr):
            for j in range(d // 16):
                rows[2, i, pl.ds(j * 16, 16)] = jnp.zeros((16,), jnp.float32)
        zstage = rows.at[2, pl.ds(0, zr)]

        def zcopy(t, issue):
            ch = t * _NS + s
            cp = pltpu.make_async_copy(
                zstage, accum.at[pl.ds(ch * zr, zr)], sem_s.at[t % 3])
            if issue:
                cp.start()
            else:
                cp.wait()

        for t in range(riters):
            if t == riters - 1:
                @pl.when(s < rchunks - (riters - 1) * _NS)
                def _():
                    zcopy(riters - 1, True)
            else:
                zcopy(t, True)

        for b in range(2):
            wait_small(src_hbm, sb, b, sem_src)
            gissue(b)

        for t in range(riters):
            if t == riters - 1:
                @pl.when(s < rchunks - (riters - 1) * _NS)
                def _():
                    zcopy(riters - 1, False)
            else:
                zcopy(t, False)
        plsc.subcore_barrier()

        ngroups = n_chunks // 3
        tail = n_chunks - ngroups * 3

        def gbody(g, carry):
            for b in range(3):
                slot(b, g * 3 + b)
            return carry
        lax.fori_loop(0, ngroups, gbody, 0)
        for t in range(tail):
            ci = ngroups * 3 + t
            b = ci % 3
            wait_g(b)
            wait_small(w_hbm, wb, b, sem_w)
            scale(b, ci)
            wait_small(dst_hbm, db, b, sem_d)
            sissue(b)
        for b in range(3):
            wait_s(b)

        if tail_e:
            toff = base0 + n_chunks * k
            pltpu.async_copy(src_hbm.at[pl.ds(toff, tail_e)],
                             sb.at[0, pl.ds(0, tail_e)], sem_src.at[0])
            pltpu.async_copy(w_hbm.at[pl.ds(toff, tail_e)],
                             wb.at[0, pl.ds(0, tail_e)], sem_w.at[0])
            pltpu.async_copy(dst_hbm.at[pl.ds(toff, tail_e)],
                             dt.at[0, pl.ds(0, tail_e)], sem_d.at[0])
            pltpu.make_async_copy(src_hbm.at[pl.ds(toff, tail_e)],
                                  sb.at[0, pl.ds(0, tail_e)],
                                  sem_src.at[0]).wait()
            pltpu.async_copy(h_hbm.at[sb.at[0, pl.ds(0, tail_e)]],
                             rows.at[0, pl.ds(0, tail_e)], sem_g.at[0])
            pltpu.make_async_copy(w_hbm.at[pl.ds(toff, tail_e)],
                                  wb.at[0, pl.ds(0, tail_e)],
                                  sem_w.at[0]).wait()
            pltpu.make_async_copy(h_hbm.at[sb.at[0, pl.ds(0, tail_e)]],
                                  rows.at[0, pl.ds(0, tail_e)],
                                  sem_g.at[0]).wait()
            w16t = wb[0, pl.ds(0, 16)]
            for l in range(tail_e):
                bct = lax.gather(
                    w16t, jnp.full((16, 1), l, jnp.int32),
                    lax.GatherDimensionNumbers(
                        offset_dims=(), collapsed_slice_dims=(0,),
                        start_index_map=(0,)),
                    slice_sizes=(1,),
                    mode=lax.GatherScatterMode.PROMISE_IN_BOUNDS)
                for j in range(d // 16):
                    rows[0, l, pl.ds(j * 16, 16)] = (
                        rows[0, l, pl.ds(j * 16, 16)] * bct)
            pltpu.make_async_copy(dst_hbm.at[pl.ds(toff, tail_e)],
                                  dt.at[0, pl.ds(0, tail_e)],
                                  sem_d.at[0]).wait()
            pltpu.async_copy(rows.at[0, pl.ds(0, tail_e)],
                             accum.at[dt.at[0]],
                             sem_s.at[0], add=True)
            pltpu.make_async_copy(rows.at[0, pl.ds(0, tail_e)],
                                  accum.at[dt.at[0]],
                                  sem_s.at[0]).wait()
        plsc.subcore_barrier()

        def ocopy(t, issue):
            ch = t * _NS + s
            off = ch * zr
            cp = pltpu.make_async_copy(
                accum.at[pl.ds(off, zr)],
                out_hbm.at[pl.ds(c * n + off, zr)], sem_g.at[t % 3])
            if issue:
                cp.start()
            else:
                cp.wait()

        for t in range(riters):
            if t == riters - 1:
                @pl.when(s < rchunks - (riters - 1) * _NS)
                def _():
                    ocopy(riters - 1, True)
            else:
                ocopy(t, True)
        for t in range(riters):
            if t == riters - 1:
                @pl.when(s < rchunks - (riters - 1) * _NS)
                def _():
                    ocopy(riters - 1, False)
            else:
                ocopy(t, False)

    return spmm


_spmm_h = _make_spmm(N, H, E, 96)


def _mm1_body(x_ref, w_ref, o_ref):
    o_ref[...] = jnp.dot(x_ref[...], w_ref[...],
                         preferred_element_type=jnp.float32)


def _mm2_body(p_ref, w_ref, o_ref):
    h = jnp.maximum(p_ref[:N] + p_ref[N:], 0.0)
    wp = jnp.concatenate(
        [w_ref[...], jnp.zeros((H, H - C), jnp.float32)], axis=1)
    o_ref[...] = jnp.dot(h, wp, preferred_element_type=jnp.float32)


def _loss_body(q_ref, lab_ref, mask_ref, w1_ref, o_ref):
    q = q_ref[:N, :C] + q_ref[N:, :C]
    lab = lab_ref[...]
    mx = jnp.max(q, axis=-1, keepdims=True)
    lse = mx + jnp.log(jnp.sum(jnp.exp(q - mx), axis=-1, keepdims=True))
    mk = mask_ref[...]
    s1 = jnp.sum(lab * (lse - q) * mk)
    s2 = jnp.sum(mk)
    l2 = 0.5 * WEIGHT_DECAY * jnp.sum(w1_ref[...] * w1_ref[...])
    o_ref[0, 0] = s1 / s2 + l2


def kernel(x, label, mask, edge_index, edge_weight, W1, W2):
    src = edge_index[0]
    dst = edge_index[1]

    h = pl.pallas_call(
        _mm1_body,
        out_shape=jax.ShapeDtypeStruct((N, H), jnp.float32),
    )(x, W1)

    p = _spmm_h(h, src, dst, edge_weight)

    g = pl.pallas_call(
        _mm2_body,
        out_shape=jax.ShapeDtypeStruct((N, H), jnp.float32),
    )(p, W2)

    q = _spmm_h(g, src, dst, edge_weight)

    loss = pl.pallas_call(
        _loss_body,
        out_shape=jax.ShapeDtypeStruct((1, 1), jnp.float32),
        out_specs=pl.BlockSpec(memory_space=pltpu.SMEM),
    )(q, label, jnp.reshape(mask, (N, 1)), W1)

    return loss[0, 0]
